# baseline (speedup 1.0000x reference)
import functools
import math

import jax
import jax.numpy as jnp
from jax import lax
from jax.experimental import pallas as pl
from jax.experimental.pallas import tpu as pltpu
from jax.experimental.pallas import tpu_sc as plsc

D_MODEL = 64
SCALE = math.sqrt(D_MODEL)

NUM_CORES = 2
NUM_SUBCORES = 16
NUM_WORKERS = NUM_CORES * NUM_SUBCORES

GROUP = 128
SPLIT = 104


@functools.cache
def _make_gather(nb, nt, vocab, d):
    b_per_w = nb // NUM_WORKERS
    assert b_per_w % 4 == 0
    g0 = SPLIT
    g1 = nt - g0
    assert g1 <= GROUP and g0 % 8 == 0 and g1 % 8 == 0
    d2 = 2 * d
    n_slots = 2 * b_per_w
    DEPTH = 8
    mesh = plsc.VectorSubcoreMesh(
        core_axis_name="c",
        subcore_axis_name="s",
        num_cores=NUM_CORES,
        num_subcores=NUM_SUBCORES,
    )

    @functools.partial(
        pl.kernel,
        out_type=jax.ShapeDtypeStruct((nb * nt, d2), jnp.float32),
        mesh=mesh,
        scratch_types=[
            pltpu.VMEM((b_per_w, g0), jnp.int32),
            pltpu.VMEM((b_per_w, g1), jnp.int32),
            pltpu.VMEM((DEPTH, g0, d), jnp.float32),
        ]
        + [pltpu.SemaphoreType.DMA] * (2 * DEPTH),
        compiler_params=pltpu.CompilerParams(use_tc_tiling_on_sc=False),
    )
    def gather_kernel(table_hbm, idxa_hbm, idxb_hbm, out_hbm, idxa_v, idxb_v,
                      rows_v, *sems):
        sg = sems[:DEPTH]
        so = sems[DEPTH:]
        wid = lax.axis_index("s") * NUM_CORES + lax.axis_index("c")
        b_base = wid * b_per_w
        pltpu.sync_copy(idxa_hbm.at[pl.ds(b_base, b_per_w)], idxa_v)
        pltpu.sync_copy(idxb_hbm.at[pl.ds(b_base, b_per_w)], idxb_v)

        def gcp(j, half, buf, sem):
            idx_ref = idxa_v if half == 0 else idxb_v
            glen = g0 if half == 0 else g1
            return pltpu.make_async_copy(
                table_hbm.at[idx_ref.at[j // 2]],
                rows_v.at[buf, pl.ds(0, glen)],
                sem,
            )

        def scale_buf(buf, half):
            glen = g0 if half == 0 else g1
            def srow(i, carry):
                r = 4 * i
                for k in range(4):
                    for c in range(0, d, 16):
                        rows_v[buf, r + k, pl.ds(c, 16)] = (
                            rows_v[buf, r + k, pl.ds(c, 16)] * SCALE
                        )
                return carry

            lax.fori_loop(0, glen // 4, srow, 0)

        def ocp(j, half, buf, sem):
            glen = g0 if half == 0 else g1
            row0 = (b_base + j // 2) * nt + (0 if half == 0 else g0)
            return pltpu.make_async_copy(
                rows_v.at[buf, pl.ds(0, glen)],
                out_hbm.at[pl.ds(row0, glen), pl.ds(0, d)],
                sem,
            )

        for p in range(DEPTH - 1):
            gcp(p, p % 2, p, sg[p]).start()
        n_outer = n_slots // DEPTH

        def body(k, carry):
            for i in range(DEPTH):
                j = DEPTH * k + i
                half = i % 2
                gcp(j, half, i, sg[i]).wait()
                scale_buf(i, half)
                ocp(j, half, i, so[i]).start()
                pb = (i - 1) % DEPTH
                phalf = (i - 1) % 2
                if i == 0:
                    @pl.when(k > 0)
                    def _():
                        ocp(j - 1, phalf, pb, so[pb]).wait()

                    gcp(j + DEPTH - 1, phalf, pb, sg[pb]).start()
                else:
                    ocp(j - 1, phalf, pb, so[pb]).wait()

                    @pl.when(k < n_outer - 1)
                    def _():
                        gcp(j + DEPTH - 1, phalf, pb, sg[pb]).start()

            return carry

        lax.fori_loop(0, n_outer, body, 0)
        ocp(n_slots - 1, 1, DEPTH - 1, so[DEPTH - 1]).wait()

    return gather_kernel


def kernel(x, table):
    b, t = x.shape
    vocab, d = table.shape
    idx = x.astype(jnp.int32)
    idx_a = lax.slice(idx, (0, 0), (b, SPLIT))
    idx_b = lax.slice(idx, (0, SPLIT), (b, t))
    out2 = _make_gather(b, t, vocab, d)(table, idx_a, idx_b)
    out3 = out2.reshape(b, t, 2 * d)
    return lax.slice(out3, (0, 0, 0), (b, t, d))
